# baseline (speedup 1.0000x reference)
import functools

import jax
import jax.numpy as jnp
from jax import lax
from jax.experimental import pallas as pl
from jax.experimental.pallas import tpu as pltpu
from jax.experimental.pallas import tpu_sc as plsc

N_STATES = 1000000
H = 64
B = 16384

_info = plsc.get_sparse_core_info()
_NC, _NS = _info.num_cores, _info.num_subcores
_NW = _NC * _NS
_BPW = B // _NW

_mesh = plsc.VectorSubcoreMesh(core_axis_name="c", subcore_axis_name="s")


@functools.partial(
    pl.kernel,
    mesh=_mesh,
    out_type=jax.ShapeDtypeStruct((B, H), jnp.float32),
    scratch_types=[
        pltpu.VMEM((_BPW,), jnp.int32),
        pltpu.SemaphoreType.DMA,
    ],
)
def _sc_gather(idx_hbm, table_hbm, out_hbm, idx_v, sem):
    wid = lax.axis_index("s") * _NC + lax.axis_index("c")
    base = wid * _BPW
    pltpu.sync_copy(idx_hbm.at[pl.ds(base, _BPW)], idx_v)

    def enq(g, carry):
        vec = idx_v[pl.ds(g * 16, 16)]
        for lane in range(16):
            r = vec[lane]
            pltpu.make_async_copy(
                table_hbm.at[pl.ds(r, 1)],
                out_hbm.at[pl.ds(base + g * 16 + lane, 1)],
                sem,
            ).start()
        return carry

    lax.fori_loop(0, _BPW // 16, enq, 0)

    def drain(i, carry):
        pltpu.make_async_copy(
            table_hbm.at[pl.ds(0, 1)], out_hbm.at[pl.ds(base + i, 1)], sem
        ).wait()
        return carry

    lax.fori_loop(0, _BPW, drain, 0)


def _tc_body(emb_ref, w_ref, b_ref, out_ref):
    h = jnp.tanh(emb_ref[...])
    w = w_ref[...]
    o0 = jnp.sum(h * w[0:1, :], axis=1, keepdims=True)
    o1 = jnp.sum(h * w[1:2, :], axis=1, keepdims=True)
    out_ref[...] = jnp.concatenate([o0, o1], axis=1) + b_ref[...]


def kernel(state_index, emb_table, lin_w, lin_b):
    idx = state_index.astype(jnp.int32)
    emb = _sc_gather(idx, emb_table)
    out = pl.pallas_call(
        _tc_body,
        out_shape=jax.ShapeDtypeStruct((B, 2), jnp.float32),
    )(emb, lin_w, lin_b.reshape(1, 2))
    return out
